# baseline (speedup 1.0000x reference)
import jax
import jax.numpy as jnp
from jax import lax
from jax.experimental import pallas as pl
from jax.experimental.pallas import tpu as pltpu
from jax.experimental.pallas import tpu_sc as plsc

N = 10000
E = 320000
F_IN = 128
HID = 16
NCLS = 32

NC = 2
NS = 16
NW = NC * NS

EW = E // NW
CH = 128
NCHF = EW // CH
TAIL = EW - NCHF * CH
NBUF = 8

_MESH = dict(core_axis_name="c", subcore_axis_name="s")
_SC_PARAMS = pltpu.CompilerParams(use_tc_tiling_on_sc=False)


def _sc_deg_body(ei_hbm, out_hbm, idx_v, ones_v, z_v, deg_sh, dsem):
    c = lax.axis_index("c")
    s = lax.axis_index("s")
    w = s * NC + c
    for i in range(CH // 16):
        ones_v[pl.ds(i * 16, 16)] = jnp.full((16,), 1.0, jnp.float32)
    for i in range(640 // 16):
        z_v[pl.ds(i * 16, 16)] = jnp.zeros((16,), jnp.float32)
    @pl.when(s < 15)
    def _():
        pltpu.sync_copy(z_v, deg_sh.at[pl.ds(s * 640, 640)])

    @pl.when(s == 15)
    def _():
        pltpu.sync_copy(z_v.at[pl.ds(0, N - 9600)], deg_sh.at[pl.ds(9600, N - 9600)])

    plsc.subcore_barrier()
    pltpu.sync_copy(ei_hbm.at[1, pl.ds(w * EW, EW)], idx_v)

    def body(j, carry):
        pltpu.async_copy(ones_v, deg_sh.at[idx_v.at[pl.ds(j * CH, CH)]], dsem,
                         add=True)
        return carry

    lax.fori_loop(0, NCHF, body, 0)
    pltpu.async_copy(ones_v.at[pl.ds(0, TAIL)],
                     deg_sh.at[idx_v.at[pl.ds(NCHF * CH, TAIL)]], dsem, add=True)

    def drain(j, carry):
        pltpu.make_async_copy(ones_v, deg_sh.at[idx_v.at[pl.ds(j * CH, CH)]],
                              dsem).wait()
        return carry

    lax.fori_loop(0, NCHF, drain, 0)
    pltpu.make_async_copy(ones_v.at[pl.ds(0, TAIL)],
                          deg_sh.at[idx_v.at[pl.ds(NCHF * CH, TAIL)]], dsem).wait()
    plsc.subcore_barrier()
    @pl.when(s < 15)
    def _():
        pltpu.sync_copy(deg_sh.at[pl.ds(s * 640, 640)], out_hbm.at[c, pl.ds(s * 640, 640)])

    @pl.when(s == 15)
    def _():
        pltpu.sync_copy(deg_sh.at[pl.ds(9600, N - 9600)], out_hbm.at[c, pl.ds(9600, N - 9600)])


_sc_deg = pl.kernel(
    _sc_deg_body,
    out_type=jax.ShapeDtypeStruct((NC, N), jnp.float32),
    mesh=plsc.VectorSubcoreMesh(**_MESH),
    compiler_params=_SC_PARAMS,
    scratch_types=[
        pltpu.VMEM((EW,), jnp.int32),
        pltpu.VMEM((CH,), jnp.float32),
        pltpu.VMEM((640,), jnp.float32),
        pltpu.VMEM_SHARED((N,), jnp.float32),
        pltpu.SemaphoreType.DMA,
    ],
)


def _make_sc_agg(F):
    ZR = 64
    NRS = N // NS

    def body(g_hbm, ei_hbm, out_hbm, si_v, di_v, rows_v, tail_v, z_v, agg_sh,
             g_sh, gsems, ssems, tsem):
        c = lax.axis_index("c")
        s = lax.axis_index("s")
        w = s * NC + c
        base = s * NRS
        pltpu.sync_copy(g_hbm.at[pl.ds(base, NRS)], g_sh.at[pl.ds(base, NRS)])
        for i in range(ZR):
            for t in range(F // 16):
                z_v[i, pl.ds(t * 16, 16)] = jnp.zeros((16,), jnp.float32)

        def zbody(j, carry):
            pltpu.sync_copy(z_v, agg_sh.at[pl.ds(base + j * ZR, ZR)])
            return carry

        lax.fori_loop(0, NRS // ZR, zbody, 0)
        pltpu.sync_copy(z_v.at[pl.ds(0, NRS % ZR)],
                        agg_sh.at[pl.ds(base + (NRS // ZR) * ZR, NRS % ZR)])
        plsc.subcore_barrier()

        pltpu.sync_copy(ei_hbm.at[0, pl.ds(w * EW, EW)], si_v)
        pltpu.sync_copy(ei_hbm.at[1, pl.ds(w * EW, EW)], di_v)

        def sidx(j):
            return si_v.at[pl.ds(j * CH, CH)]

        def didx(j):
            return di_v.at[pl.ds(j * CH, CH)]

        def issue_gather(j, b):
            pltpu.async_copy(g_sh.at[sidx(j)], rows_v.at[b], gsems.at[b])

        def wait_gather(j, b):
            pltpu.make_async_copy(g_sh.at[sidx(j)], rows_v.at[b], gsems.at[b]).wait()

        def issue_scatter(j, b):
            pltpu.async_copy(rows_v.at[b], agg_sh.at[didx(j)], ssems.at[b], add=True)

        def wait_scatter(j, b):
            pltpu.make_async_copy(rows_v.at[b], agg_sh.at[didx(j)], ssems.at[b]).wait()

        for b in range(NBUF):
            issue_gather(b, b)

        def body2(rr, carry):
            for b in range(NBUF):
                j = rr * NBUF + b

                @pl.when(j < NCHF)
                def _():
                    wait_gather(j, b)
                    issue_scatter(j, b)

                @pl.when(j + NBUF < NCHF)
                def _():
                    wait_scatter(j, b)
                    issue_gather(j + NBUF, b)

            return carry

        lax.fori_loop(0, (NCHF + NBUF - 1) // NBUF, body2, 0)
        for k in range(NBUF):
            j = NCHF - NBUF + k
            b = j % NBUF
            wait_scatter(j, b)
        pltpu.async_copy(g_sh.at[si_v.at[pl.ds(NCHF * CH, TAIL)]], tail_v, tsem).wait()
        pltpu.sync_copy(tail_v, agg_sh.at[di_v.at[pl.ds(NCHF * CH, TAIL)]], add=True)
        plsc.subcore_barrier()
        pltpu.sync_copy(agg_sh.at[pl.ds(s * NRS, NRS)],
                        out_hbm.at[c].at[pl.ds(s * NRS, NRS)])

    return pl.kernel(
        body,
        out_type=jax.ShapeDtypeStruct((NC, N, F), jnp.float32),
        mesh=plsc.VectorSubcoreMesh(**_MESH),
        compiler_params=_SC_PARAMS,
        scratch_types=[
            pltpu.VMEM((EW,), jnp.int32),
            pltpu.VMEM((EW,), jnp.int32),
            pltpu.VMEM((NBUF, CH, F), jnp.float32),
            pltpu.VMEM((TAIL, F), jnp.float32),
            pltpu.VMEM((ZR, F), jnp.float32),
            pltpu.VMEM_SHARED((N, F), jnp.float32),
            pltpu.VMEM_SHARED((N, F), jnp.float32),
            pltpu.SemaphoreType.DMA((NBUF,)),
            pltpu.SemaphoreType.DMA((NBUF,)),
            pltpu.SemaphoreType.DMA,
        ],
    )


_sc_agg16 = _make_sc_agg(HID)
_sc_agg32 = _make_sc_agg(NCLS)


_GB = 2
_BR = N // _GB


def _dinv_of(dp):
    return lax.rsqrt(1.0 + jnp.sum(dp, axis=1, keepdims=True))


def _tc_mm1_body(x_ref, w_ref, o_ref):
    o_ref[...] = jnp.dot(x_ref[...], w_ref[...], preferred_element_type=jnp.float32)


def _tc_mm1(x, w1t):
    return pl.pallas_call(
        _tc_mm1_body,
        grid=(_GB,),
        in_specs=[
            pl.BlockSpec((_BR, F_IN), lambda i: (i, 0)),
            pl.BlockSpec((F_IN, HID), lambda i: (0, 0)),
        ],
        out_specs=pl.BlockSpec((_BR, HID), lambda i: (i, 0)),
        out_shape=jax.ShapeDtypeStruct((N, HID), jnp.float32),
    )(x, w1t)


def _tc_scale_body(h_ref, dp_ref, o_ref):
    o_ref[...] = h_ref[...] * _dinv_of(dp_ref[...])


def _tc_scale(h1, degp):
    return pl.pallas_call(
        _tc_scale_body,
        grid=(_GB,),
        in_specs=[
            pl.BlockSpec((_BR, HID), lambda i: (i, 0)),
            pl.BlockSpec((_BR, NC), lambda i: (i, 0)),
        ],
        out_specs=pl.BlockSpec((_BR, HID), lambda i: (i, 0)),
        out_shape=jax.ShapeDtypeStruct((N, HID), jnp.float32),
    )(h1, degp)


def _tc_mid_body(p_ref, g_ref, dp_ref, b_ref, w_ref, o_ref):
    dinv = _dinv_of(dp_ref[...])
    a = p_ref[0] + p_ref[1] + g_ref[...]
    z = jnp.maximum(a * dinv + b_ref[...], 0.0)
    o_ref[...] = jnp.dot(z, w_ref[...], preferred_element_type=jnp.float32) * dinv


def _tc_mid(parts1, g1, degp, b1r, w2t):
    return pl.pallas_call(
        _tc_mid_body,
        grid=(_GB,),
        in_specs=[
            pl.BlockSpec((NC, _BR, HID), lambda i: (0, i, 0)),
            pl.BlockSpec((_BR, HID), lambda i: (i, 0)),
            pl.BlockSpec((_BR, NC), lambda i: (i, 0)),
            pl.BlockSpec((1, HID), lambda i: (0, 0)),
            pl.BlockSpec((HID, NCLS), lambda i: (0, 0)),
        ],
        out_specs=pl.BlockSpec((_BR, NCLS), lambda i: (i, 0)),
        out_shape=jax.ShapeDtypeStruct((N, NCLS), jnp.float32),
    )(parts1, g1, degp, b1r, w2t)


def _tc_out_body(p_ref, g_ref, dp_ref, b_ref, o_ref):
    dinv = _dinv_of(dp_ref[...])
    u = (p_ref[0] + p_ref[1] + g_ref[...]) * dinv + b_ref[...]
    m = jnp.max(u, axis=1, keepdims=True)
    sh = u - m
    o_ref[...] = sh - jnp.log(jnp.sum(jnp.exp(sh), axis=1, keepdims=True))


def _tc_out(parts2, g2, degp, b2r):
    return pl.pallas_call(
        _tc_out_body,
        grid=(_GB,),
        in_specs=[
            pl.BlockSpec((NC, _BR, NCLS), lambda i: (0, i, 0)),
            pl.BlockSpec((_BR, NCLS), lambda i: (i, 0)),
            pl.BlockSpec((_BR, NC), lambda i: (i, 0)),
            pl.BlockSpec((1, NCLS), lambda i: (0, 0)),
        ],
        out_specs=pl.BlockSpec((_BR, NCLS), lambda i: (i, 0)),
        out_shape=jax.ShapeDtypeStruct((N, NCLS), jnp.float32),
    )(parts2, g2, degp, b2r)


def kernel(x, edge_index, W1, b1, W2, b2):
    h1 = _tc_mm1(x, W1.T)
    degp = _sc_deg(edge_index).T
    g1 = _tc_scale(h1, degp)
    parts1 = _sc_agg16(g1, edge_index)
    g2 = _tc_mid(parts1, g1, degp, b1.reshape(1, HID), W2.T)
    parts2 = _sc_agg32(g2, edge_index)
    return _tc_out(parts2, g2, degp, b2.reshape(1, NCLS))


# speedup vs baseline: 1.1099x; 1.1099x over previous
import jax
import jax.numpy as jnp
from jax import lax
from jax.experimental import pallas as pl
from jax.experimental.pallas import tpu as pltpu
from jax.experimental.pallas import tpu_sc as plsc

N = 10000
E = 320000
F_IN = 128
HID = 16
NCLS = 32

NC = 2
NS = 16
NW = NC * NS

EW = E // NW
CH = 128
NCHF = EW // CH
TAIL = EW - NCHF * CH
NBUF = 8

_MESH = dict(core_axis_name="c", subcore_axis_name="s")
_SC_PARAMS = pltpu.CompilerParams(use_tc_tiling_on_sc=False)


def _sc_deg_body(ei_hbm, out_hbm, idx_v, ones_v, z_v, deg_sh, dsem):
    c = lax.axis_index("c")
    s = lax.axis_index("s")
    w = s * NC + c
    for i in range(CH // 16):
        ones_v[pl.ds(i * 16, 16)] = jnp.full((16,), 1.0, jnp.float32)
    for i in range(640 // 16):
        z_v[pl.ds(i * 16, 16)] = jnp.zeros((16,), jnp.float32)
    @pl.when(s < 15)
    def _():
        pltpu.sync_copy(z_v, deg_sh.at[pl.ds(s * 640, 640)])

    @pl.when(s == 15)
    def _():
        pltpu.sync_copy(z_v.at[pl.ds(0, N - 9600)], deg_sh.at[pl.ds(9600, N - 9600)])

    plsc.subcore_barrier()
    pltpu.sync_copy(ei_hbm.at[1, pl.ds(w * EW, EW)], idx_v)

    def body(j, carry):
        pltpu.async_copy(ones_v, deg_sh.at[idx_v.at[pl.ds(j * CH, CH)]], dsem,
                         add=True)
        return carry

    lax.fori_loop(0, NCHF, body, 0)
    pltpu.async_copy(ones_v.at[pl.ds(0, TAIL)],
                     deg_sh.at[idx_v.at[pl.ds(NCHF * CH, TAIL)]], dsem, add=True)

    def drain(j, carry):
        pltpu.make_async_copy(ones_v, deg_sh.at[idx_v.at[pl.ds(j * CH, CH)]],
                              dsem).wait()
        return carry

    lax.fori_loop(0, NCHF, drain, 0)
    pltpu.make_async_copy(ones_v.at[pl.ds(0, TAIL)],
                          deg_sh.at[idx_v.at[pl.ds(NCHF * CH, TAIL)]], dsem).wait()
    plsc.subcore_barrier()
    @pl.when(s < 15)
    def _():
        pltpu.sync_copy(deg_sh.at[pl.ds(s * 640, 640)], out_hbm.at[c, pl.ds(s * 640, 640)])

    @pl.when(s == 15)
    def _():
        pltpu.sync_copy(deg_sh.at[pl.ds(9600, N - 9600)], out_hbm.at[c, pl.ds(9600, N - 9600)])


_sc_deg = pl.kernel(
    _sc_deg_body,
    out_type=jax.ShapeDtypeStruct((NC, N), jnp.float32),
    mesh=plsc.VectorSubcoreMesh(**_MESH),
    compiler_params=_SC_PARAMS,
    scratch_types=[
        pltpu.VMEM((EW,), jnp.int32),
        pltpu.VMEM((CH,), jnp.float32),
        pltpu.VMEM((640,), jnp.float32),
        pltpu.VMEM_SHARED((N,), jnp.float32),
        pltpu.SemaphoreType.DMA,
    ],
)


def _make_sc_agg(F):
    ZR = 64
    NRS = N // NS

    def body(g_hbm, ei_hbm, out_hbm, si_v, di_v, rows_v, tail_v, z_v, agg_sh,
             gsems, ssems, tsem):
        c = lax.axis_index("c")
        s = lax.axis_index("s")
        w = s * NC + c
        base = s * NRS
        for i in range(ZR):
            for t in range(F // 16):
                z_v[i, pl.ds(t * 16, 16)] = jnp.zeros((16,), jnp.float32)

        def zbody(j, carry):
            pltpu.sync_copy(z_v, agg_sh.at[pl.ds(base + j * ZR, ZR)])
            return carry

        lax.fori_loop(0, NRS // ZR, zbody, 0)
        pltpu.sync_copy(z_v.at[pl.ds(0, NRS % ZR)],
                        agg_sh.at[pl.ds(base + (NRS // ZR) * ZR, NRS % ZR)])
        plsc.subcore_barrier()

        pltpu.sync_copy(ei_hbm.at[0, pl.ds(w * EW, EW)], si_v)
        pltpu.sync_copy(ei_hbm.at[1, pl.ds(w * EW, EW)], di_v)

        def sidx(j):
            return si_v.at[pl.ds(j * CH, CH)]

        def didx(j):
            return di_v.at[pl.ds(j * CH, CH)]

        def issue_gather(j, b):
            pltpu.async_copy(g_hbm.at[sidx(j)], rows_v.at[b], gsems.at[b])

        def wait_gather(j, b):
            pltpu.make_async_copy(g_hbm.at[sidx(j)], rows_v.at[b], gsems.at[b]).wait()

        def issue_scatter(j, b):
            pltpu.async_copy(rows_v.at[b], agg_sh.at[didx(j)], ssems.at[b], add=True)

        def wait_scatter(j, b):
            pltpu.make_async_copy(rows_v.at[b], agg_sh.at[didx(j)], ssems.at[b]).wait()

        for b in range(NBUF):
            issue_gather(b, b)

        def body2(rr, carry):
            for b in range(NBUF):
                j = rr * NBUF + b

                @pl.when(j < NCHF)
                def _():
                    wait_gather(j, b)
                    issue_scatter(j, b)

                @pl.when(j + NBUF < NCHF)
                def _():
                    wait_scatter(j, b)
                    issue_gather(j + NBUF, b)

            return carry

        lax.fori_loop(0, (NCHF + NBUF - 1) // NBUF, body2, 0)
        for k in range(NBUF):
            j = NCHF - NBUF + k
            b = j % NBUF
            wait_scatter(j, b)
        pltpu.async_copy(g_hbm.at[si_v.at[pl.ds(NCHF * CH, TAIL)]], tail_v, tsem).wait()
        pltpu.sync_copy(tail_v, agg_sh.at[di_v.at[pl.ds(NCHF * CH, TAIL)]], add=True)
        plsc.subcore_barrier()
        pltpu.sync_copy(agg_sh.at[pl.ds(s * NRS, NRS)],
                        out_hbm.at[c].at[pl.ds(s * NRS, NRS)])

    return pl.kernel(
        body,
        out_type=jax.ShapeDtypeStruct((NC, N, F), jnp.float32),
        mesh=plsc.VectorSubcoreMesh(**_MESH),
        compiler_params=_SC_PARAMS,
        scratch_types=[
            pltpu.VMEM((EW,), jnp.int32),
            pltpu.VMEM((EW,), jnp.int32),
            pltpu.VMEM((NBUF, CH, F), jnp.float32),
            pltpu.VMEM((TAIL, F), jnp.float32),
            pltpu.VMEM((ZR, F), jnp.float32),
            pltpu.VMEM_SHARED((N, F), jnp.float32),
            pltpu.SemaphoreType.DMA((NBUF,)),
            pltpu.SemaphoreType.DMA((NBUF,)),
            pltpu.SemaphoreType.DMA,
        ],
    )


_sc_agg16 = _make_sc_agg(HID)
_sc_agg32 = _make_sc_agg(NCLS)


_GB = 2
_BR = N // _GB


def _dinv_of(dp):
    return lax.rsqrt(1.0 + jnp.sum(dp, axis=1, keepdims=True))


def _tc_mm1_body(x_ref, w_ref, o_ref):
    o_ref[...] = jnp.dot(x_ref[...], w_ref[...], preferred_element_type=jnp.float32)


def _tc_mm1(x, w1t):
    return pl.pallas_call(
        _tc_mm1_body,
        grid=(_GB,),
        in_specs=[
            pl.BlockSpec((_BR, F_IN), lambda i: (i, 0)),
            pl.BlockSpec((F_IN, HID), lambda i: (0, 0)),
        ],
        out_specs=pl.BlockSpec((_BR, HID), lambda i: (i, 0)),
        out_shape=jax.ShapeDtypeStruct((N, HID), jnp.float32),
    )(x, w1t)


def _tc_scale_body(h_ref, dp_ref, o_ref):
    o_ref[...] = h_ref[...] * _dinv_of(dp_ref[...])


def _tc_scale(h1, degp):
    return pl.pallas_call(
        _tc_scale_body,
        grid=(_GB,),
        in_specs=[
            pl.BlockSpec((_BR, HID), lambda i: (i, 0)),
            pl.BlockSpec((_BR, NC), lambda i: (i, 0)),
        ],
        out_specs=pl.BlockSpec((_BR, HID), lambda i: (i, 0)),
        out_shape=jax.ShapeDtypeStruct((N, HID), jnp.float32),
    )(h1, degp)


def _tc_mid_body(p_ref, g_ref, dp_ref, b_ref, w_ref, o_ref):
    dinv = _dinv_of(dp_ref[...])
    a = p_ref[0] + p_ref[1] + g_ref[...]
    z = jnp.maximum(a * dinv + b_ref[...], 0.0)
    o_ref[...] = jnp.dot(z, w_ref[...], preferred_element_type=jnp.float32) * dinv


def _tc_mid(parts1, g1, degp, b1r, w2t):
    return pl.pallas_call(
        _tc_mid_body,
        grid=(_GB,),
        in_specs=[
            pl.BlockSpec((NC, _BR, HID), lambda i: (0, i, 0)),
            pl.BlockSpec((_BR, HID), lambda i: (i, 0)),
            pl.BlockSpec((_BR, NC), lambda i: (i, 0)),
            pl.BlockSpec((1, HID), lambda i: (0, 0)),
            pl.BlockSpec((HID, NCLS), lambda i: (0, 0)),
        ],
        out_specs=pl.BlockSpec((_BR, NCLS), lambda i: (i, 0)),
        out_shape=jax.ShapeDtypeStruct((N, NCLS), jnp.float32),
    )(parts1, g1, degp, b1r, w2t)


def _tc_out_body(p_ref, g_ref, dp_ref, b_ref, o_ref):
    dinv = _dinv_of(dp_ref[...])
    u = (p_ref[0] + p_ref[1] + g_ref[...]) * dinv + b_ref[...]
    m = jnp.max(u, axis=1, keepdims=True)
    sh = u - m
    o_ref[...] = sh - jnp.log(jnp.sum(jnp.exp(sh), axis=1, keepdims=True))


def _tc_out(parts2, g2, degp, b2r):
    return pl.pallas_call(
        _tc_out_body,
        grid=(_GB,),
        in_specs=[
            pl.BlockSpec((NC, _BR, NCLS), lambda i: (0, i, 0)),
            pl.BlockSpec((_BR, NCLS), lambda i: (i, 0)),
            pl.BlockSpec((_BR, NC), lambda i: (i, 0)),
            pl.BlockSpec((1, NCLS), lambda i: (0, 0)),
        ],
        out_specs=pl.BlockSpec((_BR, NCLS), lambda i: (i, 0)),
        out_shape=jax.ShapeDtypeStruct((N, NCLS), jnp.float32),
    )(parts2, g2, degp, b2r)


def kernel(x, edge_index, W1, b1, W2, b2):
    h1 = _tc_mm1(x, W1.T)
    degp = _sc_deg(edge_index).T
    g1 = _tc_scale(h1, degp)
    parts1 = _sc_agg16(g1, edge_index)
    g2 = _tc_mid(parts1, g1, degp, b1.reshape(1, HID), W2.T)
    parts2 = _sc_agg32(g2, edge_index)
    return _tc_out(parts2, g2, degp, b2.reshape(1, NCLS))
